# baseline (speedup 1.0000x reference)
import functools

import jax
import jax.numpy as jnp
from jax import lax
from jax.experimental import pallas as pl
from jax.experimental.pallas import tpu as pltpu

B = 128
V = 100000
W = 2048
C = (V + W - 1) // W

KS0 = 0
KS1 = 42
KS2 = KS0 ^ KS1 ^ 0x1BD11BDA
_ROTS = ((13, 15, 26, 6), (17, 29, 16, 24))
_INJECT = ((KS1, KS2, 1), (KS2, KS0, 2), (KS0, KS1, 3), (KS1, KS2, 4), (KS2, KS0, 5))

_TINY = float(jnp.finfo(jnp.float32).tiny)
_NEG_INF = float("-inf")


def _threefry_bits(cnt):
    x0 = jnp.zeros_like(cnt) + jnp.uint32(KS0)
    x1 = cnt + jnp.uint32(KS1)
    for g in range(5):
        for r in _ROTS[g % 2]:
            x0 = x0 + x1
            x1 = (x1 << jnp.uint32(r)) | (x1 >> jnp.uint32(32 - r))
            x1 = x1 ^ x0
        a, b, c = _INJECT[g]
        x0 = x0 + jnp.uint32(a)
        x1 = x1 + jnp.uint32(b) + jnp.uint32(c)
    return x0 ^ x1


def _body(logits_ref, out_ref, m_ref, i_ref):
    j = pl.program_id(0)

    @pl.when(j == 0)
    def _init():
        m_ref[...] = jnp.full_like(m_ref, jnp.float32(_NEG_INF))
        i_ref[...] = jnp.zeros_like(i_ref)

    x = logits_ref[...]
    row = lax.broadcasted_iota(jnp.int32, (B, W), 0)
    col = lax.broadcasted_iota(jnp.int32, (B, W), 1) + j * W
    cnt = (row * V + col).astype(jnp.uint32)

    bits = _threefry_bits(cnt)
    fl = lax.bitcast_convert_type(
        (bits >> jnp.uint32(9)) | jnp.uint32(0x3F800000), jnp.float32
    ) - jnp.float32(1.0)
    u = jnp.maximum(jnp.float32(_TINY), fl)
    g = -jnp.log(-jnp.log(u))
    vals = jnp.where(col < V, x + g, jnp.float32(_NEG_INF))

    bm = jnp.max(vals, axis=1, keepdims=True)
    bi = jnp.min(
        jnp.where(vals == bm, col, jnp.int32(0x7FFFFFFF)), axis=1, keepdims=True
    )

    pm = m_ref[:, 0:1]
    pi = i_ref[:, 0:1]
    better = bm > pm
    nm = jnp.where(better, bm, pm)
    ni = jnp.where(better, bi, pi)
    m_ref[...] = jnp.broadcast_to(nm, m_ref.shape)
    i_ref[...] = jnp.broadcast_to(ni, i_ref.shape)

    @pl.when(j == C - 1)
    def _fin():
        out_ref[...] = jnp.broadcast_to(ni, out_ref.shape)


@functools.partial(jax.jit, static_argnames=("interpret",))
def _sample(logits, interpret=False):
    out = pl.pallas_call(
        _body,
        grid=(C,),
        in_specs=[pl.BlockSpec((B, W), lambda j: (0, j))],
        out_specs=pl.BlockSpec((B, 128), lambda j: (0, 0)),
        out_shape=jax.ShapeDtypeStruct((B, 128), jnp.int32),
        scratch_shapes=[
            pltpu.VMEM((B, 128), jnp.float32),
            pltpu.VMEM((B, 128), jnp.int32),
        ],
        compiler_params=pltpu.CompilerParams(
            dimension_semantics=("arbitrary",),
        ),
        interpret=interpret,
    )(logits)
    return out[:, 0]


def kernel(logits):
    return _sample(logits).astype(jnp.int64)
